# baseline (speedup 1.0000x reference)
import functools

import jax
import jax.numpy as jnp
from jax import lax
from jax.experimental import pallas as pl
from jax.experimental.pallas import tpu as pltpu
from jax.experimental.pallas import tpu_sc as plsc

B = 16384
D = 32
L = 16
NC = 2
NS = 16
NW = NC * NS
BPW = B // NW
NQ = 4
QE = BPW // NQ
QG = QE // L

_mesh = plsc.VectorSubcoreMesh(core_axis_name="c", subcore_axis_name="s")


@functools.partial(
    pl.kernel,
    mesh=_mesh,
    out_type=jax.ShapeDtypeStruct((B,), jnp.float32),
    scratch_types=[
        pltpu.VMEM((BPW,), jnp.int32),
        pltpu.VMEM((BPW,), jnp.int32),
        pltpu.VMEM((NQ, QE), jnp.int32),
        pltpu.VMEM((NQ, QE), jnp.int32),
        pltpu.VMEM((QE, 128), jnp.float32),
        pltpu.VMEM((QE, 128), jnp.float32),
        pltpu.VMEM((QE, 128), jnp.float32),
        pltpu.VMEM((QE, 128), jnp.float32),
        pltpu.VMEM((D, 128), jnp.float32),
        pltpu.VMEM((L,), jnp.float32),
        pltpu.VMEM((BPW,), jnp.float32),
        pltpu.SemaphoreType.DMA,
        pltpu.SemaphoreType.DMA,
        pltpu.SemaphoreType.DMA,
        pltpu.SemaphoreType.DMA,
    ],
    compiler_params=pltpu.CompilerParams(
        needs_layout_passes=False, use_tc_tiling_on_sc=True),
)
def _cf_kernel(uid_hbm, iid_hbm, ut_hbm, it_hbm, wsp_hbm, b_hbm, out_hbm,
               uid_v, iid_v, ridx_u, ridx_i, ub0, ub1, ib0, ib1,
               wsp_v, b_v, out_v, us0, us1, is0, is1):
    wid = lax.axis_index("s") * NC + lax.axis_index("c")
    base = wid * BPW

    pltpu.sync_copy(uid_hbm.at[pl.ds(base, BPW)], uid_v)
    pltpu.sync_copy(iid_hbm.at[pl.ds(base, BPW)], iid_v)

    for c in range(NQ):
        for t in range(QE // L):
            s = pl.ds(c * QE + t * L, L)
            ridx_u[c, pl.ds(t * L, L)] = uid_v[s] >> 2
            ridx_i[c, pl.ds(t * L, L)] = iid_v[s] >> 2

    ubufs = (ub0, ub1)
    ibufs = (ib0, ib1)
    usems = (us0, us1)
    isems = (is0, is1)

    def fire(q):
        cu = pltpu.async_copy(ut_hbm.at[ridx_u.at[q]], ubufs[q % 2], usems[q % 2])
        ci = pltpu.async_copy(it_hbm.at[ridx_i.at[q]], ibufs[q % 2], isems[q % 2])
        return cu, ci

    inflight = [fire(0), fire(1)]

    pltpu.sync_copy(wsp_hbm, wsp_v)
    pltpu.sync_copy(b_hbm, b_v)

    bias = b_v[...]
    lane = lax.iota(jnp.int32, L)
    wvecs = [wsp_v[d, pl.ds(0, L)] for d in range(D)]

    for q in range(NQ):
        cu, ci = inflight[q]
        cu.wait()
        ci.wait()
        ub = ubufs[q % 2]
        ib = ibufs[q % 2]

        def group_body(gl, carry, q=q, ub=ub, ib=ib):
            idx = pl.ds(q * QE + gl * L, L)
            off_u = (uid_v[idx] & 3) << 5
            off_i = (iid_v[idx] & 3) << 5
            row = gl * L + lane
            acc = bias
            for d in range(D):
                u = plsc.load_gather(ub, [row, off_u + d])
                v = plsc.load_gather(ib, [row, off_i + d])
                acc = acc + u * v * wvecs[d]
            out_v[pl.ds(q * QE + gl * L, L)] = acc
            return carry

        lax.fori_loop(0, QG, group_body, 0)

        if q + 2 < NQ:
            inflight.append(fire(q + 2))

    pltpu.sync_copy(out_v, out_hbm.at[pl.ds(base, BPW)])


def kernel(user_id, item_id, user_table, item_table, fc_w, fc_b):
    ut4 = user_table.reshape(250000, 128)
    it4 = item_table.reshape(250000, 128)
    wsp = jnp.broadcast_to(fc_w.reshape(D, 1), (D, 128))
    b16 = jnp.broadcast_to(fc_b.reshape(1), (L,)).astype(jnp.float32)
    return _cf_kernel(user_id, item_id, ut4, it4, wsp, b16)


# speedup vs baseline: 3.9028x; 3.9028x over previous
import functools

import jax
import jax.numpy as jnp
from jax import lax
from jax.experimental import pallas as pl
from jax.experimental.pallas import tpu as pltpu
from jax.experimental.pallas import tpu_sc as plsc

B = 16384
D = 32
L = 16
NC = 2
NS = 16
NW = NC * NS
BPW = B // NW
G = BPW // L
DEPTH = 4

_mesh = plsc.VectorSubcoreMesh(core_axis_name="c", subcore_axis_name="s")


@functools.partial(
    pl.kernel,
    mesh=_mesh,
    out_type=jax.ShapeDtypeStruct((B,), jnp.float32),
    scratch_types=[
        pltpu.VMEM((BPW,), jnp.int32),
        pltpu.VMEM((BPW,), jnp.int32),
        pltpu.VMEM((DEPTH, D, 128), jnp.float32),
        pltpu.VMEM((DEPTH, D, 128), jnp.float32),
        pltpu.VMEM((D,), jnp.float32),
        pltpu.VMEM((L,), jnp.float32),
        pltpu.VMEM((BPW,), jnp.float32),
        [pltpu.SemaphoreType.DMA] * DEPTH,
        [pltpu.SemaphoreType.DMA] * DEPTH,
    ],
    compiler_params=pltpu.CompilerParams(
        needs_layout_passes=False, use_tc_tiling_on_sc=True),
)
def _cf_kernel(uid_hbm, iid_hbm, ut_hbm, it_hbm, w_hbm, b_hbm, out_hbm,
               uid_v, iid_v, bu, bi_, w_v, b_v, out_v, usems, isems):
    wid = lax.axis_index("s") * NC + lax.axis_index("c")
    base = wid * BPW

    pltpu.sync_copy(uid_hbm.at[pl.ds(base, BPW)], uid_v)
    pltpu.sync_copy(iid_hbm.at[pl.ds(base, BPW)], iid_v)
    pltpu.sync_copy(w_hbm, w_v)
    pltpu.sync_copy(b_hbm, b_v)

    bias = b_v[...]
    lane = lax.iota(jnp.int32, L)
    w0 = w_v[pl.ds(0, L)]
    w1 = w_v[pl.ds(L, L)]
    dlo = lane
    dhi = lane + L

    def fire(ru, ri, slot):
        cu = pl.multiple_of((ru >> 7) << 7, 128)
        ci = pl.multiple_of((ri >> 7) << 7, 128)
        pltpu.async_copy(ut_hbm.at[:, pl.ds(cu, 128)], bu.at[slot], usems[slot])
        pltpu.async_copy(it_hbm.at[:, pl.ds(ci, 128)], bi_.at[slot], isems[slot])

    def drain(slot):
        pltpu.make_async_copy(
            ut_hbm.at[:, pl.ds(0, 128)], bu.at[slot], usems[slot]).wait()
        pltpu.make_async_copy(
            it_hbm.at[:, pl.ds(0, 128)], bi_.at[slot], isems[slot]).wait()

    uvec0 = uid_v[pl.ds(0, L)]
    ivec0 = iid_v[pl.ds(0, L)]
    for j in range(DEPTH):
        fire(uvec0[j], ivec0[j], j)

    def group_body(g, carry):
        uvec = uid_v[pl.ds(g * L, L)]
        ivec = iid_v[pl.ds(g * L, L)]
        nbase = jnp.minimum(g + 1, G - 1) * L
        uvecn = uid_v[pl.ds(nbase, L)]
        ivecn = iid_v[pl.ds(nbase, L)]
        z = bias
        for k in range(L):
            slot = k % DEPTH
            drain(slot)
            ru = jnp.broadcast_to(uvec[k] & 127, (L,))
            ri = jnp.broadcast_to(ivec[k] & 127, (L,))
            u0 = plsc.load_gather(bu.at[slot], [dlo, ru])
            u1 = plsc.load_gather(bu.at[slot], [dhi, ru])
            v0 = plsc.load_gather(bi_.at[slot], [dlo, ri])
            v1 = plsc.load_gather(bi_.at[slot], [dhi, ri])
            t = u0 * (v0 * w0) + u1 * (v1 * w1)
            z = jnp.where(lane == k, bias + jnp.sum(t), z)
            pltpu.touch(bu)
            pltpu.touch(bi_)
            if k + DEPTH < L:

                @pl.when(jnp.bool_(True))
                def _(k=k, slot=slot, uvec=uvec, ivec=ivec):
                    fire(uvec[k + DEPTH], ivec[k + DEPTH], slot)
            else:

                @pl.when(g < G - 1)
                def _(k=k, slot=slot, uvecn=uvecn, ivecn=ivecn):
                    fire(uvecn[k + DEPTH - L], ivecn[k + DEPTH - L], slot)
        out_v[pl.ds(g * L, L)] = z
        return carry

    lax.fori_loop(0, G, group_body, 0)

    pltpu.sync_copy(out_v, out_hbm.at[pl.ds(base, BPW)])


def kernel(user_id, item_id, user_table, item_table, fc_w, fc_b):
    ut_t = user_table.T
    it_t = item_table.T
    w = fc_w.reshape(D)
    b16 = jnp.broadcast_to(fc_b.reshape(1), (L,)).astype(jnp.float32)
    return _cf_kernel(user_id, item_id, ut_t, it_t, w, b16)


# speedup vs baseline: 3.9361x; 1.0085x over previous
import functools

import jax
import jax.numpy as jnp
from jax import lax
from jax.experimental import pallas as pl
from jax.experimental.pallas import tpu as pltpu
from jax.experimental.pallas import tpu_sc as plsc

B = 16384
D = 32
L = 16
NC = 2
NS = 16
NW = NC * NS
BPW = B // NW
G = BPW // L
DEPTH = 8

_mesh = plsc.VectorSubcoreMesh(core_axis_name="c", subcore_axis_name="s")


@functools.partial(
    pl.kernel,
    mesh=_mesh,
    out_type=jax.ShapeDtypeStruct((B,), jnp.float32),
    scratch_types=[
        pltpu.VMEM((BPW,), jnp.int32),
        pltpu.VMEM((BPW,), jnp.int32),
        pltpu.VMEM((DEPTH, D, 128), jnp.float32),
        pltpu.VMEM((DEPTH, D, 128), jnp.float32),
        pltpu.VMEM((D,), jnp.float32),
        pltpu.VMEM((L,), jnp.float32),
        pltpu.VMEM((BPW,), jnp.float32),
        [pltpu.SemaphoreType.DMA] * DEPTH,
        [pltpu.SemaphoreType.DMA] * DEPTH,
    ],
    compiler_params=pltpu.CompilerParams(
        needs_layout_passes=False, use_tc_tiling_on_sc=True),
)
def _cf_kernel(uid_hbm, iid_hbm, ut_hbm, it_hbm, w_hbm, b_hbm, out_hbm,
               uid_v, iid_v, bu, bi_, w_v, b_v, out_v, usems, isems):
    wid = lax.axis_index("s") * NC + lax.axis_index("c")
    base = wid * BPW

    pltpu.sync_copy(uid_hbm.at[pl.ds(base, BPW)], uid_v)
    pltpu.sync_copy(iid_hbm.at[pl.ds(base, BPW)], iid_v)
    pltpu.sync_copy(w_hbm, w_v)
    pltpu.sync_copy(b_hbm, b_v)

    bias = b_v[...]
    lane = lax.iota(jnp.int32, L)
    w0 = w_v[pl.ds(0, L)]
    w1 = w_v[pl.ds(L, L)]
    dlo = lane
    dhi = lane + L

    def fire(ru, ri, slot):
        cu = pl.multiple_of((ru >> 7) << 7, 128)
        ci = pl.multiple_of((ri >> 7) << 7, 128)
        pltpu.async_copy(ut_hbm.at[:, pl.ds(cu, 128)], bu.at[slot], usems[slot])
        pltpu.async_copy(it_hbm.at[:, pl.ds(ci, 128)], bi_.at[slot], isems[slot])

    def drain(slot):
        pltpu.make_async_copy(
            ut_hbm.at[:, pl.ds(0, 128)], bu.at[slot], usems[slot]).wait()
        pltpu.make_async_copy(
            it_hbm.at[:, pl.ds(0, 128)], bi_.at[slot], isems[slot]).wait()

    uvec0 = uid_v[pl.ds(0, L)]
    ivec0 = iid_v[pl.ds(0, L)]
    for j in range(DEPTH):
        fire(uvec0[j], ivec0[j], j)

    def group_body(g, carry):
        uvec = uid_v[pl.ds(g * L, L)]
        ivec = iid_v[pl.ds(g * L, L)]
        nbase = jnp.minimum(g + 1, G - 1) * L
        uvecn = uid_v[pl.ds(nbase, L)]
        ivecn = iid_v[pl.ds(nbase, L)]
        z = bias
        for k in range(L):
            slot = k % DEPTH
            drain(slot)
            ru = jnp.broadcast_to(uvec[k] & 127, (L,))
            ri = jnp.broadcast_to(ivec[k] & 127, (L,))
            u0 = plsc.load_gather(bu.at[slot], [dlo, ru])
            u1 = plsc.load_gather(bu.at[slot], [dhi, ru])
            v0 = plsc.load_gather(bi_.at[slot], [dlo, ri])
            v1 = plsc.load_gather(bi_.at[slot], [dhi, ri])
            t = u0 * (v0 * w0) + u1 * (v1 * w1)
            z = jnp.where(lane == k, bias + jnp.sum(t), z)
            pltpu.touch(bu)
            pltpu.touch(bi_)
            if k + DEPTH < L:

                @pl.when(jnp.bool_(True))
                def _(k=k, slot=slot, uvec=uvec, ivec=ivec):
                    fire(uvec[k + DEPTH], ivec[k + DEPTH], slot)
            else:

                @pl.when(g < G - 1)
                def _(k=k, slot=slot, uvecn=uvecn, ivecn=ivecn):
                    fire(uvecn[k + DEPTH - L], ivecn[k + DEPTH - L], slot)
        out_v[pl.ds(g * L, L)] = z
        return carry

    lax.fori_loop(0, G, group_body, 0)

    pltpu.sync_copy(out_v, out_hbm.at[pl.ds(base, BPW)])


def kernel(user_id, item_id, user_table, item_table, fc_w, fc_b):
    ut_t = user_table.T
    it_t = item_table.T
    w = fc_w.reshape(D)
    b16 = jnp.broadcast_to(fc_b.reshape(1), (L,)).astype(jnp.float32)
    return _cf_kernel(user_id, item_id, ut_t, it_t, w, b16)


# speedup vs baseline: 3.9397x; 1.0009x over previous
import functools

import jax
import jax.numpy as jnp
from jax import lax
from jax.experimental import pallas as pl
from jax.experimental.pallas import tpu as pltpu
from jax.experimental.pallas import tpu_sc as plsc

B = 16384
D = 32
L = 16
NC = 2
NS = 16
NW = NC * NS
BPW = B // NW
G = BPW // L
DEPTH = 8

_mesh = plsc.VectorSubcoreMesh(core_axis_name="c", subcore_axis_name="s")


@functools.partial(
    pl.kernel,
    mesh=_mesh,
    out_type=jax.ShapeDtypeStruct((B,), jnp.float32),
    scratch_types=[
        pltpu.VMEM((BPW,), jnp.int32),
        pltpu.VMEM((BPW,), jnp.int32),
        pltpu.VMEM((DEPTH, D, 128), jnp.float32),
        pltpu.VMEM((DEPTH, D, 128), jnp.float32),
        pltpu.VMEM((D,), jnp.float32),
        pltpu.VMEM((L,), jnp.float32),
        pltpu.VMEM((BPW,), jnp.float32),
        [pltpu.SemaphoreType.DMA] * DEPTH,
        [pltpu.SemaphoreType.DMA] * DEPTH,
    ],
    compiler_params=pltpu.CompilerParams(
        needs_layout_passes=False, use_tc_tiling_on_sc=True),
)
def _cf_kernel(uid_hbm, iid_hbm, ut_hbm, it_hbm, w_hbm, b_hbm, out_hbm,
               uid_v, iid_v, bu, bi_, w_v, b_v, out_v, usems, isems):
    wid = lax.axis_index("s") * NC + lax.axis_index("c")
    base = wid * BPW

    pltpu.sync_copy(uid_hbm.at[pl.ds(base, BPW)], uid_v)
    pltpu.sync_copy(iid_hbm.at[pl.ds(base, BPW)], iid_v)
    pltpu.sync_copy(w_hbm, w_v)
    pltpu.sync_copy(b_hbm, b_v)

    bias = b_v[...]
    lane = lax.iota(jnp.int32, L)
    w0 = w_v[pl.ds(0, L)]
    w1 = w_v[pl.ds(L, L)]
    dlo = lane
    dhi = lane + L

    def fire(ru, ri, slot):
        cu = pl.multiple_of((ru >> 7) << 7, 128)
        ci = pl.multiple_of((ri >> 7) << 7, 128)
        pltpu.async_copy(ut_hbm.at[:, pl.ds(cu, 128)], bu.at[slot], usems[slot])
        pltpu.async_copy(it_hbm.at[:, pl.ds(ci, 128)], bi_.at[slot], isems[slot])

    def drain(slot):
        pltpu.make_async_copy(
            ut_hbm.at[:, pl.ds(0, 128)], bu.at[slot], usems[slot]).wait()
        pltpu.make_async_copy(
            it_hbm.at[:, pl.ds(0, 128)], bi_.at[slot], isems[slot]).wait()

    uvec0 = uid_v[pl.ds(0, L)]
    ivec0 = iid_v[pl.ds(0, L)]
    for j in range(DEPTH):
        fire(uvec0[j], ivec0[j], j)

    def group_body(g, carry):
        uvec = uid_v[pl.ds(g * L, L)]
        ivec = iid_v[pl.ds(g * L, L)]
        nbase = jnp.minimum(g + 1, G - 1) * L
        uvecn = uid_v[pl.ds(nbase, L)]
        ivecn = iid_v[pl.ds(nbase, L)]
        z = bias
        for k in range(L):
            slot = k % DEPTH
            drain(slot)
            ru = jnp.broadcast_to(uvec[k] & 127, (L,))
            ri = jnp.broadcast_to(ivec[k] & 127, (L,))
            u0 = plsc.load_gather(bu.at[slot], [dlo, ru])
            u1 = plsc.load_gather(bu.at[slot], [dhi, ru])
            v0 = plsc.load_gather(bi_.at[slot], [dlo, ri])
            v1 = plsc.load_gather(bi_.at[slot], [dhi, ri])
            t = u0 * (v0 * w0) + u1 * (v1 * w1)
            z = jnp.where(lane == k, bias + jnp.sum(t), z)
            pltpu.touch(bu)
            pltpu.touch(bi_)
            if k + DEPTH < L:
                fire(uvec[k + DEPTH], ivec[k + DEPTH], slot)
            else:

                @pl.when(g < G - 1)
                def _(k=k, slot=slot, uvecn=uvecn, ivecn=ivecn):
                    fire(uvecn[k + DEPTH - L], ivecn[k + DEPTH - L], slot)
        out_v[pl.ds(g * L, L)] = z
        return carry

    lax.fori_loop(0, G, group_body, 0)

    pltpu.sync_copy(out_v, out_hbm.at[pl.ds(base, BPW)])


def kernel(user_id, item_id, user_table, item_table, fc_w, fc_b):
    ut_t = user_table.T
    it_t = item_table.T
    w = fc_w.reshape(D)
    b16 = jnp.broadcast_to(fc_b.reshape(1), (L,)).astype(jnp.float32)
    return _cf_kernel(user_id, item_id, ut_t, it_t, w, b16)
